# Initial kernel scaffold; baseline (speedup 1.0000x reference)
import jax
import jax.numpy as jnp
from jax.experimental import pallas as pl


def kernel(x, edge_index, W1, b1, beta2, W2, b2):
    raise NotImplementedError("write your pallas kernel here")


# speedup vs baseline: 52.2354x; 52.2354x over previous
import functools

import jax
import jax.numpy as jnp
from jax import lax
from jax.experimental import pallas as pl
from jax.experimental.pallas import tpu as pltpu
from jax.experimental.pallas import tpu_sc as plsc

N = 50000
N_PAD = 50048
E = 1600000
F_IN = 128
H = 16
C = 16
EPS = 1e-12

NW = 32
SUB = 128
SUBS_PER_SUPER = 40
SUPERS = 10
EDGES_PER_W = SUB * SUBS_PER_SUPER * SUPERS
E_PAD = EDGES_PER_W * NW
ROWS_PER_TILE = N_PAD // 16
ZCHUNKS = N_PAD // SUB


GRID = 8
BR = N_PAD // GRID


def _prep1_body(x_ref, w1_ref, b1_ref, combo_ref):
    h = jnp.dot(x_ref[...], w1_ref[...], preferred_element_type=jnp.float32)
    h = jnp.maximum(h + b1_ref[...], 0.0)
    rows = pl.program_id(0) * BR + lax.broadcasted_iota(jnp.int32, (BR, 1), 0)
    h = jnp.where(rows < N, h, 0.0)
    nrm = jnp.sqrt(jnp.sum(h * h, axis=1, keepdims=True))
    hn = h / jnp.maximum(nrm, EPS)
    combo_ref[...] = jnp.concatenate([hn, h], axis=1)


def _mid_body(p_ref, beta_ref, combo_ref):
    p = p_ref[0] + p_ref[1]
    den = jnp.maximum(p[:, 16:17], EPS)
    h = p[:, 0:16] / den
    rows = pl.program_id(0) * BR + lax.broadcasted_iota(jnp.int32, (BR, 1), 0)
    h = jnp.where(rows < N, h, 0.0)
    nrm = jnp.sqrt(jnp.sum(h * h, axis=1, keepdims=True))
    hn = h / jnp.maximum(nrm, EPS)
    sb = jnp.sqrt(beta_ref[0, 0])
    combo_ref[...] = jnp.concatenate([hn * sb, h], axis=1)


def _out_body(p_ref, w2_ref, b2_ref, out_ref):
    p = p_ref[0] + p_ref[1]
    den = jnp.maximum(p[:, 16:17], EPS)
    h = p[:, 0:16] / den
    logits = jnp.dot(h, w2_ref[...], preferred_element_type=jnp.float32)
    logits = logits + b2_ref[...]
    m = jnp.max(logits, axis=1, keepdims=True)
    lse = jnp.log(jnp.sum(jnp.exp(logits - m), axis=1, keepdims=True)) + m
    out_ref[...] = logits - lse


def _full(shape):
    return pl.BlockSpec(shape, lambda i: tuple(0 for _ in shape))


def _prep1(xp, W1, b1):
    return pl.pallas_call(
        _prep1_body,
        grid=(GRID,),
        in_specs=[
            pl.BlockSpec((BR, F_IN), lambda i: (i, 0)),
            _full((F_IN, H)),
            _full((1, H)),
        ],
        out_specs=pl.BlockSpec((BR, 32), lambda i: (i, 0)),
        out_shape=jax.ShapeDtypeStruct((N_PAD, 32), jnp.float32),
    )(xp, W1, b1)


def _mid(partials, beta2):
    return pl.pallas_call(
        _mid_body,
        grid=(GRID,),
        in_specs=[
            pl.BlockSpec((2, BR, 32), lambda i: (0, i, 0)),
            _full((1, 1)),
        ],
        out_specs=pl.BlockSpec((BR, 32), lambda i: (i, 0)),
        out_shape=jax.ShapeDtypeStruct((N_PAD, 32), jnp.float32),
    )(partials, beta2)


def _out(partials, W2, b2):
    return pl.pallas_call(
        _out_body,
        grid=(GRID,),
        in_specs=[
            pl.BlockSpec((2, BR, 32), lambda i: (0, i, 0)),
            _full((H, C)),
            _full((1, C)),
        ],
        out_specs=pl.BlockSpec((BR, C), lambda i: (i, 0)),
        out_shape=jax.ShapeDtypeStruct((N_PAD, C), jnp.float32),
    )(partials, W2, b2)


_MESH = plsc.VectorSubcoreMesh(core_axis_name="c", subcore_axis_name="s")


@functools.partial(
    pl.kernel,
    out_type=jax.ShapeDtypeStruct((2, N_PAD, 32), jnp.float32),
    mesh=_MESH,
    scratch_types=[
        pltpu.VMEM((SUBS_PER_SUPER, SUB), jnp.int32),
        pltpu.VMEM((SUBS_PER_SUPER, SUB), jnp.int32),
        pltpu.VMEM((SUB, 32), jnp.float32),
        pltpu.VMEM((SUB, 32), jnp.float32),
        pltpu.VMEM((SUB, 32), jnp.float32),
        pltpu.VMEM((SUB, 32), jnp.float32),
        pltpu.VMEM_SHARED((N_PAD, 32), jnp.float32),
        pltpu.SemaphoreType.DMA,
    ],
    compiler_params=pltpu.CompilerParams(
        needs_layout_passes=False, use_tc_tiling_on_sc=False),
)
def _agnn_pass(combo_hbm, src_hbm, dst_hbm, out_hbm,
               src_v, dst_v, rows_s, rows_d, upd_v, zero_v, acc_sh, sem_g):
    c = lax.axis_index("c")
    s = lax.axis_index("s")
    wid = s * 2 + c

    z16 = jnp.zeros((16,), jnp.float32)

    @pl.loop(0, SUB)
    def _zfill(i):
        zero_v[i, 0:16] = z16
        zero_v[i, 16:32] = z16

    nz = jnp.where(s < (ZCHUNKS % 16), ZCHUNKS // 16 + 1, ZCHUNKS // 16)

    @pl.loop(0, nz)
    def _zero(i):
        k = s + i * 16
        pltpu.sync_copy(zero_v, acc_sh.at[pl.ds(k * SUB, SUB)])

    plsc.subcore_barrier()

    rbase = wid * (EDGES_PER_W // SUB)

    @pl.loop(0, SUPERS)
    def _super(sp):
        roff = rbase + sp * SUBS_PER_SUPER
        pltpu.sync_copy(src_hbm.at[pl.ds(roff, SUBS_PER_SUPER)], src_v)
        pltpu.sync_copy(dst_hbm.at[pl.ds(roff, SUBS_PER_SUPER)], dst_v)

        @pl.loop(0, SUBS_PER_SUPER)
        def _sub(j):
            g1 = pltpu.async_copy(combo_hbm.at[src_v.at[j]], rows_s, sem_g)
            g2 = pltpu.async_copy(combo_hbm.at[dst_v.at[j]], rows_d, sem_g)
            g1.wait()
            g2.wait()

            @plsc.parallel_loop(0, SUB, unroll=8)
            def _edge(e):
                cs = rows_s[e, 0:16]
                cd = rows_d[e, 0:16]
                hs = rows_s[e, 16:32]
                t = jnp.sum(cs * cd)
                w = jnp.exp(jnp.broadcast_to(t, (16,)))
                upd_v[e, 0:16] = w * hs
                upd_v[e, 16:32] = w

            pltpu.sync_copy(upd_v, acc_sh.at[dst_v.at[j]], add=True)

    plsc.subcore_barrier()
    pltpu.sync_copy(
        acc_sh.at[pl.ds(s * ROWS_PER_TILE, ROWS_PER_TILE)],
        out_hbm.at[c, pl.ds(s * ROWS_PER_TILE, ROWS_PER_TILE)],
    )


@jax.jit
def _run(x, edge_index, W1, b1, beta2, W2, b2):
    src = edge_index[0].astype(jnp.int32)
    dst = edge_index[1].astype(jnp.int32)
    npad = E_PAD - E
    k = jnp.arange(npad, dtype=jnp.int32)
    pad_idx = N + (k % (N_PAD - N))
    srcp = jnp.concatenate([src, pad_idx]).reshape(E_PAD // SUB, SUB)
    dstp = jnp.concatenate([dst, pad_idx]).reshape(E_PAD // SUB, SUB)

    xp = jnp.pad(x, ((0, N_PAD - N), (0, 0)))
    combo1 = _prep1(xp, W1, b1.reshape(1, H))
    p1 = _agnn_pass(combo1, srcp, dstp)
    combo2 = _mid(p1, beta2.reshape(1, 1))
    p2 = _agnn_pass(combo2, srcp, dstp)
    out = _out(p2, W2, b2.reshape(1, C))
    return out[:N]


def kernel(x, edge_index, W1, b1, beta2, W2, b2):
    return _run(x, edge_index, W1, b1, beta2, W2, b2)


# speedup vs baseline: 85.0995x; 1.6292x over previous
import functools

import jax
import jax.numpy as jnp
from jax import lax
from jax.experimental import pallas as pl
from jax.experimental.pallas import tpu as pltpu
from jax.experimental.pallas import tpu_sc as plsc

N = 50000
N_PAD = 50048
E = 1600000
F_IN = 128
H = 16
C = 16
EPS = 1e-12

NW = 32
SUB = 128
SUBS_PER_SUPER = 40
SUPERS = 10
EDGES_PER_W = SUB * SUBS_PER_SUPER * SUPERS
E_PAD = EDGES_PER_W * NW
ROWS_PER_TILE = N_PAD // 16
ZCHUNKS = N_PAD // SUB


GRID = 8
BR = N_PAD // GRID


def _prep1_body(x_ref, w1_ref, b1_ref, combo_ref, g_ref):
    h = jnp.dot(x_ref[...], w1_ref[...], preferred_element_type=jnp.float32)
    h = jnp.maximum(h + b1_ref[...], 0.0)
    rows = pl.program_id(0) * BR + lax.broadcasted_iota(jnp.int32, (BR, 1), 0)
    h = jnp.where(rows < N, h, 0.0)
    nrm = jnp.sqrt(jnp.sum(h * h, axis=1, keepdims=True))
    hn = h / jnp.maximum(nrm, EPS)
    combo_ref[...] = jnp.concatenate([hn, h], axis=1)
    g_ref[...] = hn


def _mid_body(p_ref, beta_ref, combo_ref, g_ref):
    p = p_ref[0] + p_ref[1]
    den = jnp.maximum(p[:, 16:17], EPS)
    h = p[:, 0:16] / den
    rows = pl.program_id(0) * BR + lax.broadcasted_iota(jnp.int32, (BR, 1), 0)
    h = jnp.where(rows < N, h, 0.0)
    nrm = jnp.sqrt(jnp.sum(h * h, axis=1, keepdims=True))
    hn = h / jnp.maximum(nrm, EPS)
    sb = jnp.sqrt(beta_ref[0, 0])
    hnb = hn * sb
    combo_ref[...] = jnp.concatenate([hnb, h], axis=1)
    g_ref[...] = hnb


def _out_body(p_ref, w2_ref, b2_ref, out_ref):
    p = p_ref[0] + p_ref[1]
    den = jnp.maximum(p[:, 16:17], EPS)
    h = p[:, 0:16] / den
    logits = jnp.dot(h, w2_ref[...], preferred_element_type=jnp.float32)
    logits = logits + b2_ref[...]
    m = jnp.max(logits, axis=1, keepdims=True)
    lse = jnp.log(jnp.sum(jnp.exp(logits - m), axis=1, keepdims=True)) + m
    out_ref[...] = logits - lse


def _full(shape):
    return pl.BlockSpec(shape, lambda i: tuple(0 for _ in shape))


def _prep1(xp, W1, b1):
    return pl.pallas_call(
        _prep1_body,
        grid=(GRID,),
        in_specs=[
            pl.BlockSpec((BR, F_IN), lambda i: (i, 0)),
            _full((F_IN, H)),
            _full((1, H)),
        ],
        out_specs=[pl.BlockSpec((BR, 32), lambda i: (i, 0)),
                   pl.BlockSpec((BR, H), lambda i: (i, 0))],
        out_shape=[jax.ShapeDtypeStruct((N_PAD, 32), jnp.float32),
                   jax.ShapeDtypeStruct((N_PAD, H), jnp.float32)],
    )(xp, W1, b1)


def _mid(partials, beta2):
    return pl.pallas_call(
        _mid_body,
        grid=(GRID,),
        in_specs=[
            pl.BlockSpec((2, BR, 32), lambda i: (0, i, 0)),
            _full((1, 1)),
        ],
        out_specs=[pl.BlockSpec((BR, 32), lambda i: (i, 0)),
                   pl.BlockSpec((BR, H), lambda i: (i, 0))],
        out_shape=[jax.ShapeDtypeStruct((N_PAD, 32), jnp.float32),
                   jax.ShapeDtypeStruct((N_PAD, H), jnp.float32)],
    )(partials, beta2)


def _out(partials, W2, b2):
    return pl.pallas_call(
        _out_body,
        grid=(GRID,),
        in_specs=[
            pl.BlockSpec((2, BR, 32), lambda i: (0, i, 0)),
            _full((H, C)),
            _full((1, C)),
        ],
        out_specs=pl.BlockSpec((BR, C), lambda i: (i, 0)),
        out_shape=jax.ShapeDtypeStruct((N_PAD, C), jnp.float32),
    )(partials, W2, b2)


_MESH = plsc.VectorSubcoreMesh(core_axis_name="c", subcore_axis_name="s")

SUBS_TOTAL = EDGES_PER_W // SUB
RPS = 10
SUPERS2 = SUBS_TOTAL // RPS


@functools.partial(
    pl.kernel,
    out_type=jax.ShapeDtypeStruct((2, N_PAD, 32), jnp.float32),
    mesh=_MESH,
    scratch_types=[
        pltpu.VMEM((2, RPS, SUB), jnp.int32),
        pltpu.VMEM((2, RPS, SUB), jnp.int32),
        pltpu.VMEM((2, SUB, 32), jnp.float32),
        pltpu.VMEM((2, SUB, H), jnp.float32),
        pltpu.VMEM((2, SUB, 32), jnp.float32),
        pltpu.VMEM_SHARED((N_PAD, 32), jnp.float32),
        pltpu.SemaphoreType.DMA,
        pltpu.SemaphoreType.DMA,
        pltpu.SemaphoreType.DMA,
        pltpu.SemaphoreType.DMA,
        pltpu.SemaphoreType.DMA,
        pltpu.SemaphoreType.DMA,
    ],
    compiler_params=pltpu.CompilerParams(
        needs_layout_passes=False, use_tc_tiling_on_sc=False),
)
def _agnn_pass(combo_hbm, g_hbm, src_hbm, dst_hbm, out_hbm,
               src_v, dst_v, rows_s, rows_d, upd_v, acc_sh,
               isem0, isem1, gsem0, gsem1, ssem0, ssem1):
    c = lax.axis_index("c")
    s = lax.axis_index("s")
    wid = s * 2 + c
    isems = [isem0, isem1]
    gsems = [gsem0, gsem1]
    ssems = [ssem0, ssem1]

    z16 = jnp.zeros((16,), jnp.float32)

    @pl.loop(0, SUB)
    def _zfill(i):
        upd_v[0, i, 0:16] = z16
        upd_v[0, i, 16:32] = z16

    nz = jnp.where(s < (ZCHUNKS % 16), ZCHUNKS // 16 + 1, ZCHUNKS // 16)

    @pl.loop(0, nz)
    def _zero(i):
        k = s + i * 16
        pltpu.sync_copy(upd_v.at[0], acc_sh.at[pl.ds(k * SUB, SUB)])

    rbase = wid * SUBS_TOTAL
    pltpu.async_copy(src_hbm.at[pl.ds(rbase, RPS)], src_v.at[0], isem0)
    pltpu.async_copy(dst_hbm.at[pl.ds(rbase, RPS)], dst_v.at[0], isem0)
    plsc.subcore_barrier()

    def _super_body(sp, q):
        sq = src_v.at[q]
        dq = dst_v.at[q]
        pltpu.make_async_copy(
            src_hbm.at[pl.ds(rbase, RPS)], sq, isems[q]).wait()
        pltpu.make_async_copy(
            dst_hbm.at[pl.ds(rbase, RPS)], dq, isems[q]).wait()

        pltpu.async_copy(combo_hbm.at[sq.at[0]], rows_s.at[0], gsem0)
        pltpu.async_copy(g_hbm.at[dq.at[0]], rows_d.at[0], gsem0)

        @pl.loop(0, RPS, step=2)
        def _pipe(r0):
            for par in range(2):
                r = r0 + par
                cur, nxt = par, 1 - par
                rs = rows_s.at[cur]
                rd = rows_d.at[cur]
                up = upd_v.at[cur]

                def _prefetch():
                    pltpu.async_copy(
                        combo_hbm.at[sq.at[r + 1]], rows_s.at[nxt], gsems[nxt])
                    pltpu.async_copy(
                        g_hbm.at[dq.at[r + 1]], rows_d.at[nxt], gsems[nxt])

                if par == 0:
                    _prefetch()

                    @pl.when(jnp.logical_and(r0 == 2, sp < SUPERS2 - 1))
                    def _nexti():
                        roff = rbase + (sp + 1) * RPS
                        pltpu.async_copy(
                            src_hbm.at[pl.ds(roff, RPS)], src_v.at[1 - q],
                            isems[1 - q])
                        pltpu.async_copy(
                            dst_hbm.at[pl.ds(roff, RPS)], dst_v.at[1 - q],
                            isems[1 - q])
                else:
                    pl.when(r0 < RPS - 2)(_prefetch)

                pltpu.make_async_copy(
                    combo_hbm.at[sq.at[r]], rs, gsems[cur]).wait()
                pltpu.make_async_copy(
                    g_hbm.at[dq.at[r]], rd, gsems[cur]).wait()

                j = sp * RPS + r

                @pl.when(j >= 2)
                def _drain():
                    pltpu.make_async_copy(
                        up, acc_sh.at[dq.at[0]], ssems[cur]).wait()

                @plsc.parallel_loop(0, SUB, unroll=8)
                def _edge(e):
                    cs = rs[e, 0:16]
                    cd = rd[e, 0:16]
                    hs = rs[e, 16:32]
                    t = jnp.sum(cs * cd)
                    w = jnp.exp(jnp.broadcast_to(t, (16,)))
                    up[e, 0:16] = w * hs
                    up[e, 16:32] = w

                pltpu.async_copy(up, acc_sh.at[dq.at[r]], ssems[cur], add=True)

    @pl.loop(0, SUPERS2, step=2)
    def _super(sp0):
        _super_body(sp0, 0)
        _super_body(sp0 + 1, 1)

    pltpu.make_async_copy(upd_v.at[0], acc_sh.at[dst_v.at[0, 0]], ssem0).wait()
    pltpu.make_async_copy(upd_v.at[1], acc_sh.at[dst_v.at[0, 0]], ssem1).wait()

    plsc.subcore_barrier()
    pltpu.sync_copy(
        acc_sh.at[pl.ds(s * ROWS_PER_TILE, ROWS_PER_TILE)],
        out_hbm.at[c, pl.ds(s * ROWS_PER_TILE, ROWS_PER_TILE)],
    )


@jax.jit
def _run(x, edge_index, W1, b1, beta2, W2, b2):
    src = edge_index[0].astype(jnp.int32)
    dst = edge_index[1].astype(jnp.int32)
    npad = E_PAD - E
    k = jnp.arange(npad, dtype=jnp.int32)
    pad_idx = N + (k % (N_PAD - N))
    srcp = jnp.concatenate([src, pad_idx]).reshape(E_PAD // SUB, SUB)
    dstp = jnp.concatenate([dst, pad_idx]).reshape(E_PAD // SUB, SUB)

    xp = jnp.pad(x, ((0, N_PAD - N), (0, 0)))
    combo1, g1 = _prep1(xp, W1, b1.reshape(1, H))
    p1 = _agnn_pass(combo1, g1, srcp, dstp)
    combo2, g2 = _mid(p1, beta2.reshape(1, 1))
    p2 = _agnn_pass(combo2, g2, srcp, dstp)
    out = _out(p2, W2, b2.reshape(1, C))
    return out[:N]


def kernel(x, edge_index, W1, b1, beta2, W2, b2):
    return _run(x, edge_index, W1, b1, beta2, W2, b2)


# speedup vs baseline: 86.9927x; 1.0222x over previous
import functools

import jax
import jax.numpy as jnp
from jax import lax
from jax.experimental import pallas as pl
from jax.experimental.pallas import tpu as pltpu
from jax.experimental.pallas import tpu_sc as plsc

N = 50000
N_PAD = 50048
E = 1600000
F_IN = 128
H = 16
C = 16
EPS = 1e-12

NW = 32
SUB = 128
SUBS_PER_SUPER = 40
SUPERS = 10
EDGES_PER_W = SUB * SUBS_PER_SUPER * SUPERS
E_PAD = EDGES_PER_W * NW
ROWS_PER_TILE = N_PAD // 16
ZCHUNKS = N_PAD // SUB


GRID = 8
BR = N_PAD // GRID


def _prep1_body(x_ref, w1_ref, b1_ref, combo_ref, g_ref):
    h = jnp.dot(x_ref[...], w1_ref[...], preferred_element_type=jnp.float32)
    h = jnp.maximum(h + b1_ref[...], 0.0)
    rows = pl.program_id(0) * BR + lax.broadcasted_iota(jnp.int32, (BR, 1), 0)
    h = jnp.where(rows < N, h, 0.0)
    nrm = jnp.sqrt(jnp.sum(h * h, axis=1, keepdims=True))
    hn = h / jnp.maximum(nrm, EPS)
    combo_ref[...] = jnp.concatenate([hn, h], axis=1)
    g_ref[...] = hn


def _mid_body(p_ref, beta_ref, combo_ref, g_ref):
    p = p_ref[0] + p_ref[1]
    den = jnp.maximum(p[:, 16:17], EPS)
    h = p[:, 0:16] / den
    rows = pl.program_id(0) * BR + lax.broadcasted_iota(jnp.int32, (BR, 1), 0)
    h = jnp.where(rows < N, h, 0.0)
    nrm = jnp.sqrt(jnp.sum(h * h, axis=1, keepdims=True))
    hn = h / jnp.maximum(nrm, EPS)
    sb = jnp.sqrt(beta_ref[0, 0])
    hnb = hn * sb
    combo_ref[...] = jnp.concatenate([hnb, h], axis=1)
    g_ref[...] = hnb


OBR = 5000


def _out_body(p_ref, w2_ref, b2_ref, out_ref):
    p = p_ref[0] + p_ref[1]
    den = jnp.maximum(p[:, 16:17], EPS)
    h = p[:, 0:16] / den
    logits = jnp.dot(h, w2_ref[...], preferred_element_type=jnp.float32)
    logits = logits + b2_ref[...]
    m = jnp.max(logits, axis=1, keepdims=True)
    lse = jnp.log(jnp.sum(jnp.exp(logits - m), axis=1, keepdims=True)) + m
    out_ref[...] = logits - lse


def _full(shape):
    return pl.BlockSpec(shape, lambda i: tuple(0 for _ in shape))


def _prep1(xp, W1, b1):
    return pl.pallas_call(
        _prep1_body,
        grid=(GRID,),
        in_specs=[
            pl.BlockSpec((BR, F_IN), lambda i: (i, 0)),
            _full((F_IN, H)),
            _full((1, H)),
        ],
        out_specs=[pl.BlockSpec((BR, 32), lambda i: (i, 0)),
                   pl.BlockSpec((BR, H), lambda i: (i, 0))],
        out_shape=[jax.ShapeDtypeStruct((N_PAD, 32), jnp.float32),
                   jax.ShapeDtypeStruct((N_PAD, H), jnp.float32)],
    )(xp, W1, b1)


def _mid(partials, beta2):
    return pl.pallas_call(
        _mid_body,
        grid=(GRID,),
        in_specs=[
            pl.BlockSpec((2, BR, 32), lambda i: (0, i, 0)),
            _full((1, 1)),
        ],
        out_specs=[pl.BlockSpec((BR, 32), lambda i: (i, 0)),
                   pl.BlockSpec((BR, H), lambda i: (i, 0))],
        out_shape=[jax.ShapeDtypeStruct((N_PAD, 32), jnp.float32),
                   jax.ShapeDtypeStruct((N_PAD, H), jnp.float32)],
    )(partials, beta2)


def _out(partials, W2, b2):
    return pl.pallas_call(
        _out_body,
        grid=(N // OBR,),
        in_specs=[
            pl.BlockSpec((2, OBR, 32), lambda i: (0, i, 0)),
            _full((H, C)),
            _full((1, C)),
        ],
        out_specs=pl.BlockSpec((OBR, C), lambda i: (i, 0)),
        out_shape=jax.ShapeDtypeStruct((N, C), jnp.float32),
    )(partials, W2, b2)


_MESH = plsc.VectorSubcoreMesh(core_axis_name="c", subcore_axis_name="s")

SUBS_TOTAL = EDGES_PER_W // SUB
RPS = 10
SUPERS2 = SUBS_TOTAL // RPS


@functools.partial(
    pl.kernel,
    out_type=jax.ShapeDtypeStruct((2, N_PAD, 32), jnp.float32),
    mesh=_MESH,
    scratch_types=[
        pltpu.VMEM((2, RPS, SUB), jnp.int32),
        pltpu.VMEM((2, RPS, SUB), jnp.int32),
        pltpu.VMEM((2, SUB, 32), jnp.float32),
        pltpu.VMEM((2, SUB, H), jnp.float32),
        pltpu.VMEM((2, SUB, 32), jnp.float32),
        pltpu.VMEM_SHARED((N_PAD, 32), jnp.float32),
        pltpu.SemaphoreType.DMA,
        pltpu.SemaphoreType.DMA,
        pltpu.SemaphoreType.DMA,
        pltpu.SemaphoreType.DMA,
        pltpu.SemaphoreType.DMA,
        pltpu.SemaphoreType.DMA,
    ],
    compiler_params=pltpu.CompilerParams(
        needs_layout_passes=False, use_tc_tiling_on_sc=False),
)
def _agnn_pass(combo_hbm, g_hbm, src_hbm, dst_hbm, out_hbm,
               src_v, dst_v, rows_s, rows_d, upd_v, acc_sh,
               isem0, isem1, gsem0, gsem1, ssem0, ssem1):
    c = lax.axis_index("c")
    s = lax.axis_index("s")
    wid = s * 2 + c
    isems = [isem0, isem1]
    gsems = [gsem0, gsem1]
    ssems = [ssem0, ssem1]

    z16 = jnp.zeros((16,), jnp.float32)

    @pl.loop(0, SUB)
    def _zfill(i):
        upd_v[0, i, 0:16] = z16
        upd_v[0, i, 16:32] = z16

    nz = jnp.where(s < (ZCHUNKS % 16), ZCHUNKS // 16 + 1, ZCHUNKS // 16)

    @pl.loop(0, nz)
    def _zero(i):
        k = s + i * 16
        pltpu.sync_copy(upd_v.at[0], acc_sh.at[pl.ds(k * SUB, SUB)])

    rbase = wid * SUBS_TOTAL
    pltpu.async_copy(src_hbm.at[pl.ds(rbase, RPS)], src_v.at[0], isem0)
    pltpu.async_copy(dst_hbm.at[pl.ds(rbase, RPS)], dst_v.at[0], isem0)
    plsc.subcore_barrier()

    def _super_body(sp, q):
        sq = src_v.at[q]
        dq = dst_v.at[q]
        pltpu.make_async_copy(
            src_hbm.at[pl.ds(rbase, RPS)], sq, isems[q]).wait()
        pltpu.make_async_copy(
            dst_hbm.at[pl.ds(rbase, RPS)], dq, isems[q]).wait()

        pltpu.async_copy(combo_hbm.at[sq.at[0]], rows_s.at[0], gsem0)
        pltpu.async_copy(g_hbm.at[dq.at[0]], rows_d.at[0], gsem0)

        @pl.loop(0, RPS, step=2)
        def _pipe(r0):
            for par in range(2):
                r = r0 + par
                cur, nxt = par, 1 - par
                rs = rows_s.at[cur]
                rd = rows_d.at[cur]
                up = upd_v.at[cur]

                def _prefetch():
                    pltpu.async_copy(
                        combo_hbm.at[sq.at[r + 1]], rows_s.at[nxt], gsems[nxt])
                    pltpu.async_copy(
                        g_hbm.at[dq.at[r + 1]], rows_d.at[nxt], gsems[nxt])

                if par == 0:
                    _prefetch()

                    @pl.when(jnp.logical_and(r0 == 2, sp < SUPERS2 - 1))
                    def _nexti():
                        roff = rbase + (sp + 1) * RPS
                        pltpu.async_copy(
                            src_hbm.at[pl.ds(roff, RPS)], src_v.at[1 - q],
                            isems[1 - q])
                        pltpu.async_copy(
                            dst_hbm.at[pl.ds(roff, RPS)], dst_v.at[1 - q],
                            isems[1 - q])
                else:
                    pl.when(r0 < RPS - 2)(_prefetch)

                pltpu.make_async_copy(
                    combo_hbm.at[sq.at[r]], rs, gsems[cur]).wait()
                pltpu.make_async_copy(
                    g_hbm.at[dq.at[r]], rd, gsems[cur]).wait()

                j = sp * RPS + r

                @pl.when(j >= 2)
                def _drain():
                    pltpu.make_async_copy(
                        up, acc_sh.at[dq.at[0]], ssems[cur]).wait()

                @plsc.parallel_loop(0, SUB, unroll=16)
                def _edge(e):
                    cs = rs[e, 0:16]
                    cd = rd[e, 0:16]
                    hs = rs[e, 16:32]
                    t = jnp.sum(cs * cd)
                    w = jnp.exp(jnp.broadcast_to(t, (16,)))
                    up[e, 0:16] = w * hs
                    up[e, 16:32] = w

                pltpu.async_copy(up, acc_sh.at[dq.at[r]], ssems[cur], add=True)

    @pl.loop(0, SUPERS2, step=2)
    def _super(sp0):
        _super_body(sp0, 0)
        _super_body(sp0 + 1, 1)

    pltpu.make_async_copy(upd_v.at[0], acc_sh.at[dst_v.at[0, 0]], ssem0).wait()
    pltpu.make_async_copy(upd_v.at[1], acc_sh.at[dst_v.at[0, 0]], ssem1).wait()

    plsc.subcore_barrier()
    pltpu.sync_copy(
        acc_sh.at[pl.ds(s * ROWS_PER_TILE, ROWS_PER_TILE)],
        out_hbm.at[c, pl.ds(s * ROWS_PER_TILE, ROWS_PER_TILE)],
    )


@jax.jit
def _run(x, edge_index, W1, b1, beta2, W2, b2):
    src = edge_index[0].astype(jnp.int32)
    dst = edge_index[1].astype(jnp.int32)
    npad = E_PAD - E
    k = jnp.arange(npad, dtype=jnp.int32)
    pad_idx = N + (k % (N_PAD - N))
    srcp = jnp.concatenate([src, pad_idx]).reshape(E_PAD // SUB, SUB)
    dstp = jnp.concatenate([dst, pad_idx]).reshape(E_PAD // SUB, SUB)

    xp = jnp.pad(x, ((0, N_PAD - N), (0, 0)))
    combo1, g1 = _prep1(xp, W1, b1.reshape(1, H))
    p1 = _agnn_pass(combo1, g1, srcp, dstp)
    combo2, g2 = _mid(p1, beta2.reshape(1, 1))
    p2 = _agnn_pass(combo2, g2, srcp, dstp)
    return _out(p2, W2, b2.reshape(1, C))


def kernel(x, edge_index, W1, b1, beta2, W2, b2):
    return _run(x, edge_index, W1, b1, beta2, W2, b2)


# speedup vs baseline: 91.8803x; 1.0562x over previous
import functools

import jax
import jax.numpy as jnp
from jax import lax
from jax.experimental import pallas as pl
from jax.experimental.pallas import tpu as pltpu
from jax.experimental.pallas import tpu_sc as plsc

N = 50000
N_PAD = 50048
E = 1600000
F_IN = 128
H = 16
C = 16
EPS = 1e-12

NW = 32
SUB = 128
SUBS_PER_SUPER = 40
SUPERS = 10
EDGES_PER_W = SUB * SUBS_PER_SUPER * SUPERS
E_PAD = EDGES_PER_W * NW
ROWS_PER_TILE = N_PAD // 16
ZCHUNKS = N_PAD // SUB


GRID = 8
BR = N_PAD // GRID


def _prep1_body(x_ref, w1_ref, b1_ref, combo_ref, g_ref):
    h = jnp.dot(x_ref[...], w1_ref[...], preferred_element_type=jnp.float32)
    h = jnp.maximum(h + b1_ref[...], 0.0)
    rows = pl.program_id(0) * BR + lax.broadcasted_iota(jnp.int32, (BR, 1), 0)
    h = jnp.where(rows < N, h, 0.0)
    nrm = jnp.sqrt(jnp.sum(h * h, axis=1, keepdims=True))
    hn = h / jnp.maximum(nrm, EPS)
    combo_ref[...] = jnp.concatenate([hn, h], axis=1)
    g_ref[...] = hn


def _mid_body(p_ref, beta_ref, combo_ref, g_ref):
    p = p_ref[0] + p_ref[1]
    den = jnp.maximum(p[:, 16:17], EPS)
    h = p[:, 0:16] / den
    rows = pl.program_id(0) * BR + lax.broadcasted_iota(jnp.int32, (BR, 1), 0)
    h = jnp.where(rows < N, h, 0.0)
    nrm = jnp.sqrt(jnp.sum(h * h, axis=1, keepdims=True))
    hn = h / jnp.maximum(nrm, EPS)
    sb = jnp.sqrt(beta_ref[0, 0])
    hnb = hn * sb
    combo_ref[...] = jnp.concatenate([hnb, h], axis=1)
    g_ref[...] = hnb


OBR = 5000


def _out_body(p_ref, w2_ref, b2_ref, out_ref):
    p = p_ref[0] + p_ref[1]
    den = jnp.maximum(p[:, 16:17], EPS)
    h = p[:, 0:16] / den
    logits = jnp.dot(h, w2_ref[...], preferred_element_type=jnp.float32)
    logits = logits + b2_ref[...]
    m = jnp.max(logits, axis=1, keepdims=True)
    lse = jnp.log(jnp.sum(jnp.exp(logits - m), axis=1, keepdims=True)) + m
    out_ref[...] = logits - lse


EBLK = 65536
EROWS = E // SUB
EROWS_PAD = E_PAD // SUB


def _edges_body(e_ref, s_ref, d_ref):
    ev = e_ref[...]
    s = ev[0].reshape(EBLK // SUB, SUB)
    d = ev[1].reshape(EBLK // SUB, SUB)
    rows = pl.program_id(0) * (EBLK // SUB) + lax.broadcasted_iota(
        jnp.int32, (EBLK // SUB, SUB), 0)
    lanes = lax.broadcasted_iota(jnp.int32, (EBLK // SUB, SUB), 1)
    pad = N + (rows * SUB + lanes) % (N_PAD - N)
    real = rows < EROWS
    s_ref[...] = jnp.where(real, s, pad)
    d_ref[...] = jnp.where(real, d, pad)


def _edges(edge_index):
    nblk = EROWS_PAD * SUB // EBLK
    return pl.pallas_call(
        _edges_body,
        grid=(nblk,),
        in_specs=[pl.BlockSpec((2, EBLK), lambda i: (0, i))],
        out_specs=[pl.BlockSpec((EBLK // SUB, SUB), lambda i: (i, 0)),
                   pl.BlockSpec((EBLK // SUB, SUB), lambda i: (i, 0))],
        out_shape=[jax.ShapeDtypeStruct((EROWS_PAD, SUB), jnp.int32),
                   jax.ShapeDtypeStruct((EROWS_PAD, SUB), jnp.int32)],
    )(edge_index)


def _full(shape):
    return pl.BlockSpec(shape, lambda i: tuple(0 for _ in shape))


def _prep1(xp, W1, b1):
    return pl.pallas_call(
        _prep1_body,
        grid=(GRID,),
        in_specs=[
            pl.BlockSpec((BR, F_IN), lambda i: (i, 0)),
            _full((F_IN, H)),
            _full((1, H)),
        ],
        out_specs=[pl.BlockSpec((BR, 32), lambda i: (i, 0)),
                   pl.BlockSpec((BR, H), lambda i: (i, 0))],
        out_shape=[jax.ShapeDtypeStruct((N_PAD, 32), jnp.float32),
                   jax.ShapeDtypeStruct((N_PAD, H), jnp.float32)],
    )(xp, W1, b1)


def _mid(partials, beta2):
    return pl.pallas_call(
        _mid_body,
        grid=(GRID,),
        in_specs=[
            pl.BlockSpec((2, BR, 32), lambda i: (0, i, 0)),
            _full((1, 1)),
        ],
        out_specs=[pl.BlockSpec((BR, 32), lambda i: (i, 0)),
                   pl.BlockSpec((BR, H), lambda i: (i, 0))],
        out_shape=[jax.ShapeDtypeStruct((N_PAD, 32), jnp.float32),
                   jax.ShapeDtypeStruct((N_PAD, H), jnp.float32)],
    )(partials, beta2)


def _out(partials, W2, b2):
    return pl.pallas_call(
        _out_body,
        grid=(N // OBR,),
        in_specs=[
            pl.BlockSpec((2, OBR, 32), lambda i: (0, i, 0)),
            _full((H, C)),
            _full((1, C)),
        ],
        out_specs=pl.BlockSpec((OBR, C), lambda i: (i, 0)),
        out_shape=jax.ShapeDtypeStruct((N, C), jnp.float32),
    )(partials, W2, b2)


_MESH = plsc.VectorSubcoreMesh(core_axis_name="c", subcore_axis_name="s")

SUBS_TOTAL = EDGES_PER_W // SUB
RPS = 10
SUPERS2 = SUBS_TOTAL // RPS


@functools.partial(
    pl.kernel,
    out_type=jax.ShapeDtypeStruct((2, N_PAD, 32), jnp.float32),
    mesh=_MESH,
    scratch_types=[
        pltpu.VMEM((2, RPS, SUB), jnp.int32),
        pltpu.VMEM((2, RPS, SUB), jnp.int32),
        pltpu.VMEM((2, SUB, 32), jnp.float32),
        pltpu.VMEM((2, SUB, H), jnp.float32),
        pltpu.VMEM((2, SUB, 32), jnp.float32),
        pltpu.VMEM_SHARED((N_PAD, 32), jnp.float32),
        pltpu.SemaphoreType.DMA,
        pltpu.SemaphoreType.DMA,
        pltpu.SemaphoreType.DMA,
        pltpu.SemaphoreType.DMA,
        pltpu.SemaphoreType.DMA,
        pltpu.SemaphoreType.DMA,
    ],
    compiler_params=pltpu.CompilerParams(
        needs_layout_passes=False, use_tc_tiling_on_sc=False),
)
def _agnn_pass(combo_hbm, g_hbm, src_hbm, dst_hbm, out_hbm,
               src_v, dst_v, rows_s, rows_d, upd_v, acc_sh,
               isem0, isem1, gsem0, gsem1, ssem0, ssem1):
    c = lax.axis_index("c")
    s = lax.axis_index("s")
    wid = s * 2 + c
    isems = [isem0, isem1]
    gsems = [gsem0, gsem1]
    ssems = [ssem0, ssem1]

    z16 = jnp.zeros((16,), jnp.float32)

    @pl.loop(0, SUB)
    def _zfill(i):
        upd_v[0, i, 0:16] = z16
        upd_v[0, i, 16:32] = z16

    nz = jnp.where(s < (ZCHUNKS % 16), ZCHUNKS // 16 + 1, ZCHUNKS // 16)

    @pl.loop(0, nz)
    def _zero(i):
        k = s + i * 16
        pltpu.sync_copy(upd_v.at[0], acc_sh.at[pl.ds(k * SUB, SUB)])

    rbase = wid * SUBS_TOTAL
    pltpu.async_copy(src_hbm.at[pl.ds(rbase, RPS)], src_v.at[0], isem0)
    pltpu.async_copy(dst_hbm.at[pl.ds(rbase, RPS)], dst_v.at[0], isem0)
    plsc.subcore_barrier()

    def _super_body(sp, q):
        sq = src_v.at[q]
        dq = dst_v.at[q]
        pltpu.make_async_copy(
            src_hbm.at[pl.ds(rbase, RPS)], sq, isems[q]).wait()
        pltpu.make_async_copy(
            dst_hbm.at[pl.ds(rbase, RPS)], dq, isems[q]).wait()

        pltpu.async_copy(combo_hbm.at[sq.at[0]], rows_s.at[0], gsem0)
        pltpu.async_copy(g_hbm.at[dq.at[0]], rows_d.at[0], gsem0)

        @pl.loop(0, RPS, step=2)
        def _pipe(r0):
            for par in range(2):
                r = r0 + par
                cur, nxt = par, 1 - par
                rs = rows_s.at[cur]
                rd = rows_d.at[cur]
                up = upd_v.at[cur]

                def _prefetch():
                    pltpu.async_copy(
                        combo_hbm.at[sq.at[r + 1]], rows_s.at[nxt], gsems[nxt])
                    pltpu.async_copy(
                        g_hbm.at[dq.at[r + 1]], rows_d.at[nxt], gsems[nxt])

                if par == 0:
                    _prefetch()

                    @pl.when(jnp.logical_and(r0 == 2, sp < SUPERS2 - 1))
                    def _nexti():
                        roff = rbase + (sp + 1) * RPS
                        pltpu.async_copy(
                            src_hbm.at[pl.ds(roff, RPS)], src_v.at[1 - q],
                            isems[1 - q])
                        pltpu.async_copy(
                            dst_hbm.at[pl.ds(roff, RPS)], dst_v.at[1 - q],
                            isems[1 - q])
                else:
                    pl.when(r0 < RPS - 2)(_prefetch)

                pltpu.make_async_copy(
                    combo_hbm.at[sq.at[r]], rs, gsems[cur]).wait()
                pltpu.make_async_copy(
                    g_hbm.at[dq.at[r]], rd, gsems[cur]).wait()

                j = sp * RPS + r

                @pl.when(j >= 2)
                def _drain():
                    pltpu.make_async_copy(
                        up, acc_sh.at[dq.at[0]], ssems[cur]).wait()

                @plsc.parallel_loop(0, SUB, unroll=16)
                def _edge(e):
                    cs = rs[e, 0:16]
                    cd = rd[e, 0:16]
                    hs = rs[e, 16:32]
                    t = jnp.sum(cs * cd)
                    w = jnp.exp(jnp.broadcast_to(t, (16,)))
                    up[e, 0:16] = w * hs
                    up[e, 16:32] = w

                pltpu.async_copy(up, acc_sh.at[dq.at[r]], ssems[cur], add=True)

    @pl.loop(0, SUPERS2, step=2)
    def _super(sp0):
        _super_body(sp0, 0)
        _super_body(sp0 + 1, 1)

    pltpu.make_async_copy(upd_v.at[0], acc_sh.at[dst_v.at[0, 0]], ssem0).wait()
    pltpu.make_async_copy(upd_v.at[1], acc_sh.at[dst_v.at[0, 0]], ssem1).wait()

    plsc.subcore_barrier()
    pltpu.sync_copy(
        acc_sh.at[pl.ds(s * ROWS_PER_TILE, ROWS_PER_TILE)],
        out_hbm.at[c, pl.ds(s * ROWS_PER_TILE, ROWS_PER_TILE)],
    )


@jax.jit
def _run(x, edge_index, W1, b1, beta2, W2, b2):
    srcp, dstp = _edges(edge_index.astype(jnp.int32))

    xp = jnp.pad(x, ((0, N_PAD - N), (0, 0)))
    combo1, g1 = _prep1(xp, W1, b1.reshape(1, H))
    p1 = _agnn_pass(combo1, g1, srcp, dstp)
    combo2, g2 = _mid(p1, beta2.reshape(1, 1))
    p2 = _agnn_pass(combo2, g2, srcp, dstp)
    return _out(p2, W2, b2.reshape(1, C))


def kernel(x, edge_index, W1, b1, beta2, W2, b2):
    return _run(x, edge_index, W1, b1, beta2, W2, b2)
